# Initial kernel scaffold; baseline (speedup 1.0000x reference)
import jax
import jax.numpy as jnp
from jax.experimental import pallas as pl


def kernel(x, edge_index, timestep, W, a, temporal_weight):
    raise NotImplementedError("write your pallas kernel here")


# speedup vs baseline: 43.0020x; 43.0020x over previous
import functools

import jax
import jax.numpy as jnp
from jax import lax
from jax.experimental import pallas as pl
from jax.experimental.pallas import tpu as pltpu
from jax.experimental.pallas import tpu_sc as plsc

N_NODES = 10000
N_EDGES = 320000
IN_FEAT = 128
OUT_FEAT = 16
HEADS = 8
HF = HEADS * OUT_FEAT
AUXW = 16
ACCW = HF + AUXW

NC = 2
NS = 16
NW = NC * NS
EDGES_PER_TILE = N_EDGES // NW
CHUNK = 80
NCHUNK = EDGES_PER_TILE // CHUNK
ROWS_PER_TILE = N_NODES // NS


def _prep_body(x_ref, wf_ref, as_ref, ad_ref, t_ref, h_ref, auxs_ref, auxd_ref):
    xb = x_ref[...]
    hb = jnp.dot(xb, wf_ref[...], preferred_element_type=jnp.float32)
    h_ref[...] = hb
    tb = t_ref[...]
    trow = jnp.concatenate(
        [jnp.zeros((1, 8), jnp.float32), jnp.ones((1, 8), jnp.float32)], axis=1)
    auxs_ref[...] = jnp.dot(hb, as_ref[...], preferred_element_type=jnp.float32) + tb * trow
    auxd_ref[...] = jnp.dot(hb, ad_ref[...], preferred_element_type=jnp.float32) + tb * trow


def _lane_bcast(v, idx):
    dnums = lax.GatherDimensionNumbers(
        offset_dims=(), collapsed_slice_dims=(0,), start_index_map=(0,))
    return lax.gather(v, idx[:, None], dnums, (1,),
                      mode=lax.GatherScatterMode.PROMISE_IN_BOUNDS)


def _edge_body(src_hbm, dst_hbm, auxs_hbm, auxd_hbm, h_hbm, zeros_hbm, ntw_hbm,
               out_hbm, accum, sidx, didx, rows_s, rows_d, hrows, msg, ntw_v,
               sem_s, sem_d, sem_h):
    core = lax.axis_index("c")
    sid = lax.axis_index("s")
    wid = sid * NC + core
    base = wid * EDGES_PER_TILE

    pltpu.sync_copy(zeros_hbm, accum.at[pl.ds(sid * ROWS_PER_TILE, ROWS_PER_TILE)])
    pltpu.sync_copy(ntw_hbm, ntw_v)
    plsc.subcore_barrier()

    ntw = ntw_v[...]
    bidx = [jnp.full((16,), j, jnp.int32) for j in range(9)]

    def chunk_body(c, _):
        off = base + c * CHUNK
        pltpu.sync_copy(src_hbm.at[pl.ds(off, CHUNK)], sidx)
        pltpu.sync_copy(dst_hbm.at[pl.ds(off, CHUNK)], didx)
        cp_s = pltpu.async_copy(auxs_hbm.at[sidx], rows_s, sem_s)
        cp_d = pltpu.async_copy(auxd_hbm.at[didx], rows_d, sem_d)
        cp_h = pltpu.async_copy(h_hbm.at[sidx], hrows, sem_h)
        cp_s.wait()
        cp_d.wait()
        cp_h.wait()

        def edge_body(i, _):
            vs = rows_s[i, :]
            vd = rows_d[i, :]
            vfac = jnp.exp(jnp.abs(vs - vd) * ntw)
            fac = _lane_bcast(vfac, bidx[8])
            e = (vs + vd) * fac
            e = jnp.maximum(e, 0.2 * e)
            ee = jnp.exp(e)
            msg[i, pl.ds(HF, 16)] = ee
            for j in range(HEADS):
                bj = _lane_bcast(ee, bidx[j])
                msg[i, pl.ds(j * 16, 16)] = hrows[i, pl.ds(j * 16, 16)] * bj
            return 0

        lax.fori_loop(0, CHUNK, edge_body, 0)
        pltpu.sync_copy(msg, accum.at[didx], add=True)
        return 0

    lax.fori_loop(0, NCHUNK, chunk_body, 0)
    plsc.subcore_barrier()
    pltpu.sync_copy(accum.at[pl.ds(sid * ROWS_PER_TILE, ROWS_PER_TILE)],
                    out_hbm.at[core, pl.ds(sid * ROWS_PER_TILE, ROWS_PER_TILE)])


def _final_body(p0_ref, p1_ref, sel_ref, out_ref):
    s = p0_ref[...] + p1_ref[...]
    num = s[:, 0:HF]
    den = s[:, HF:HF + HEADS]
    denb = jnp.dot(den, sel_ref[...], preferred_element_type=jnp.float32)
    out_ref[...] = num / (denb + 1e-8)


def kernel(x, edge_index, timestep, W, a, temporal_weight):
    f32 = jnp.float32
    wf = W.transpose(1, 0, 2).reshape(IN_FEAT, HF)
    eye = jnp.eye(HEADS, dtype=f32)
    as_mat = (a[:, None, :OUT_FEAT] * eye[:, :, None]).transpose(0, 2, 1).reshape(HF, HEADS)
    ad_mat = (a[:, None, OUT_FEAT:] * eye[:, :, None]).transpose(0, 2, 1).reshape(HF, HEADS)
    as_mat = jnp.concatenate([as_mat, jnp.zeros((HF, 8), f32)], axis=1)
    ad_mat = jnp.concatenate([ad_mat, jnp.zeros((HF, 8), f32)], axis=1)
    tf = timestep.astype(f32).reshape(N_NODES, 1)

    R = 1000
    grid = N_NODES // R
    h, aux_s, aux_d = pl.pallas_call(
        _prep_body,
        grid=(grid,),
        in_specs=[
            pl.BlockSpec((R, IN_FEAT), lambda i: (i, 0)),
            pl.BlockSpec((IN_FEAT, HF), lambda i: (0, 0)),
            pl.BlockSpec((IN_FEAT, AUXW), lambda i: (0, 0)),
            pl.BlockSpec((IN_FEAT, AUXW), lambda i: (0, 0)),
            pl.BlockSpec((R, 1), lambda i: (i, 0)),
        ],
        out_specs=[
            pl.BlockSpec((R, HF), lambda i: (i, 0)),
            pl.BlockSpec((R, AUXW), lambda i: (i, 0)),
            pl.BlockSpec((R, AUXW), lambda i: (i, 0)),
        ],
        out_shape=[
            jax.ShapeDtypeStruct((N_NODES, HF), f32),
            jax.ShapeDtypeStruct((N_NODES, AUXW), f32),
            jax.ShapeDtypeStruct((N_NODES, AUXW), f32),
        ],
    )(x, wf, as_mat, ad_mat, tf)

    src = edge_index[0]
    dst = edge_index[1]
    zeros = jnp.zeros((ROWS_PER_TILE, ACCW), f32)
    ntw = jnp.full((16,), -temporal_weight, f32)

    mesh = plsc.VectorSubcoreMesh(core_axis_name="c", subcore_axis_name="s")
    partial = pl.kernel(
        _edge_body,
        out_type=jax.ShapeDtypeStruct((NC, N_NODES, ACCW), f32),
        mesh=mesh,
        compiler_params=pltpu.CompilerParams(use_tc_tiling_on_sc=False),
        scratch_types=[
            pltpu.VMEM_SHARED((N_NODES, ACCW), f32),
            pltpu.VMEM((CHUNK,), jnp.int32),
            pltpu.VMEM((CHUNK,), jnp.int32),
            pltpu.VMEM((CHUNK, AUXW), f32),
            pltpu.VMEM((CHUNK, AUXW), f32),
            pltpu.VMEM((CHUNK, HF), f32),
            pltpu.VMEM((CHUNK, ACCW), f32),
            pltpu.VMEM((16,), f32),
            pltpu.SemaphoreType.DMA,
            pltpu.SemaphoreType.DMA,
            pltpu.SemaphoreType.DMA,
        ],
    )(src, dst, aux_s, aux_d, h, zeros, ntw)

    sel = (eye[:, :, None] * jnp.ones((1, 1, OUT_FEAT), f32)).reshape(HEADS, HF)
    out = pl.pallas_call(
        _final_body,
        grid=(grid,),
        in_specs=[
            pl.BlockSpec((R, ACCW), lambda i: (i, 0)),
            pl.BlockSpec((R, ACCW), lambda i: (i, 0)),
            pl.BlockSpec((HEADS, HF), lambda i: (0, 0)),
        ],
        out_specs=pl.BlockSpec((R, HF), lambda i: (i, 0)),
        out_shape=jax.ShapeDtypeStruct((N_NODES, HF), f32),
    )(partial[0], partial[1], sel)
    return out


# speedup vs baseline: 101.1984x; 2.3533x over previous
import jax
import jax.numpy as jnp
from jax import lax
from jax.experimental import pallas as pl
from jax.experimental.pallas import tpu as pltpu
from jax.experimental.pallas import tpu_sc as plsc

N_NODES = 10000
N_EDGES = 320000
IN_FEAT = 128
OUT_FEAT = 16
HEADS = 8
HF = HEADS * OUT_FEAT
AUXW = 16
ACCW = HF + AUXW

NC = 2
NS = 16
NW = NC * NS
EDGES_PER_TILE = N_EDGES // NW
CHUNK = 40
NCHUNK = EDGES_PER_TILE // CHUNK
ROWS_PER_TILE = N_NODES // NS


def _prep_body(x_ref, wf_ref, as_ref, ad_ref, t_ref, hext_ref, auxd_ref):
    xb = x_ref[...]
    hb = jnp.dot(xb, wf_ref[...], preferred_element_type=jnp.float32)
    tb = t_ref[...]
    trow = jnp.concatenate(
        [jnp.zeros((1, 8), jnp.float32), jnp.ones((1, 8), jnp.float32)], axis=1)
    auxs = jnp.dot(hb, as_ref[...], preferred_element_type=jnp.float32) + tb * trow
    hext_ref[...] = jnp.concatenate([hb, auxs], axis=1)
    auxd_ref[...] = jnp.dot(hb, ad_ref[...], preferred_element_type=jnp.float32) + tb * trow


def _lane_bcast(v, idx):
    dnums = lax.GatherDimensionNumbers(
        offset_dims=(), collapsed_slice_dims=(0,), start_index_map=(0,))
    return lax.gather(v, idx[:, None], dnums, (1,),
                      mode=lax.GatherScatterMode.PROMISE_IN_BOUNDS)


def _edge_body(src_hbm, dst_hbm, hext_hbm, auxd_hbm, zeros_hbm, ntw_hbm,
               out_hbm, accum, sidx, didx, hsrc, rows_d, msg, ntw_v,
               sem_i, sem_g, sem_sc):
    core = lax.axis_index("c")
    sid = lax.axis_index("s")
    wid = sid * NC + core

    pltpu.sync_copy(zeros_hbm, accum.at[pl.ds(sid * ROWS_PER_TILE, ROWS_PER_TILE)])
    pltpu.sync_copy(ntw_hbm, ntw_v)
    plsc.subcore_barrier()

    ntw = ntw_v[...]
    bidx = [jnp.full((16,), j, jnp.int32) for j in range(9)]

    def fire_idx(c, q):
        pltpu.make_async_copy(src_hbm.at[wid, c], sidx.at[q], sem_i.at[q]).start()
        pltpu.make_async_copy(dst_hbm.at[wid, c], didx.at[q], sem_i.at[q]).start()

    def wait_idx(c, q):
        pltpu.make_async_copy(src_hbm.at[wid, c], sidx.at[q], sem_i.at[q]).wait()
        pltpu.make_async_copy(dst_hbm.at[wid, c], didx.at[q], sem_i.at[q]).wait()

    def fire_gathers(b, q):
        pltpu.make_async_copy(hext_hbm.at[sidx.at[q]], hsrc.at[b], sem_g.at[b]).start()
        pltpu.make_async_copy(auxd_hbm.at[didx.at[q]], rows_d.at[b], sem_g.at[b]).start()

    def wait_gathers(b, q):
        pltpu.make_async_copy(hext_hbm.at[sidx.at[q]], hsrc.at[b], sem_g.at[b]).wait()
        pltpu.make_async_copy(auxd_hbm.at[didx.at[q]], rows_d.at[b], sem_g.at[b]).wait()

    def fire_scatter(b, q):
        pltpu.make_async_copy(msg.at[b], accum.at[didx.at[q]], sem_sc.at[b]).start(add=True)

    def wait_scatter(b, q):
        pltpu.make_async_copy(msg.at[b], accum.at[didx.at[q]], sem_sc.at[b]).wait()

    def compute(b):
        def edge_body(i, _):
            vs = hsrc[b, i, pl.ds(HF, 16)]
            vd = rows_d[b, i, :]
            vfac = jnp.exp(jnp.abs(vs - vd) * ntw)
            fac = _lane_bcast(vfac, bidx[8])
            e = (vs + vd) * fac
            e = jnp.maximum(e, 0.2 * e)
            ee = jnp.exp(e)
            msg[b, i, pl.ds(HF, 16)] = ee
            for j in range(HEADS):
                bj = _lane_bcast(ee, bidx[j])
                msg[b, i, pl.ds(j * 16, 16)] = hsrc[b, i, pl.ds(j * 16, 16)] * bj
            return 0

        lax.fori_loop(0, CHUNK, edge_body, 0)

    fire_idx(0, 0)
    fire_idx(1, 1)
    wait_idx(0, 0)
    fire_gathers(0, 0)
    wait_idx(1, 1); fire_gathers(1, 1); fire_idx(2, 2)
    wait_gathers(0, 0); compute(0); fire_scatter(0, 0)
    wait_idx(2, 2); fire_gathers(0, 2); fire_idx(3, 3)
    wait_gathers(1, 1); compute(1); fire_scatter(1, 1)
    wait_idx(3, 3); fire_gathers(1, 3); wait_scatter(0, 0); fire_idx(4, 0)
    wait_gathers(0, 2); compute(0); fire_scatter(0, 2)
    wait_idx(4, 0); fire_gathers(0, 0); wait_scatter(1, 1); fire_idx(5, 1)
    wait_gathers(1, 3); compute(1); fire_scatter(1, 3)

    def quad_body(p, _):
        for k in range(4):
            c = 4 * p + k
            b = k % 2
            bn = 1 - b
            q = k
            q1 = (k + 1) % 4
            q2 = (k + 2) % 4
            wait_idx(c + 1, q1)
            fire_gathers(bn, q1)
            wait_scatter(b, q2)
            fire_idx(c + 2, q2)
            wait_gathers(b, q)
            compute(b)
            fire_scatter(b, q)
        return 0

    lax.fori_loop(1, NCHUNK // 4, quad_body, 0)

    wait_idx(NCHUNK - 1, 1); fire_gathers(1, 1); wait_scatter(0, 2)
    wait_gathers(0, 0); compute(0); fire_scatter(0, 0)
    wait_scatter(1, 3)
    wait_gathers(1, 1); compute(1); fire_scatter(1, 1)
    wait_scatter(0, 0)
    wait_scatter(1, 1)

    plsc.subcore_barrier()
    pltpu.sync_copy(accum.at[pl.ds(sid * ROWS_PER_TILE, ROWS_PER_TILE)],
                    out_hbm.at[core, pl.ds(sid * ROWS_PER_TILE, ROWS_PER_TILE)])


def _final_body(p0_ref, p1_ref, sel_ref, out_ref):
    s = p0_ref[...] + p1_ref[...]
    num = s[:, 0:HF]
    den = s[:, HF:HF + HEADS]
    denb = jnp.dot(den, sel_ref[...], preferred_element_type=jnp.float32)
    out_ref[...] = num / (denb + 1e-8)


def kernel(x, edge_index, timestep, W, a, temporal_weight):
    f32 = jnp.float32
    wf = W.transpose(1, 0, 2).reshape(IN_FEAT, HF)
    eye = jnp.eye(HEADS, dtype=f32)
    as_mat = (a[:, None, :OUT_FEAT] * eye[:, :, None]).transpose(0, 2, 1).reshape(HF, HEADS)
    ad_mat = (a[:, None, OUT_FEAT:] * eye[:, :, None]).transpose(0, 2, 1).reshape(HF, HEADS)
    as_mat = jnp.concatenate([as_mat, jnp.zeros((HF, 8), f32)], axis=1)
    ad_mat = jnp.concatenate([ad_mat, jnp.zeros((HF, 8), f32)], axis=1)
    tf = timestep.astype(f32).reshape(N_NODES, 1)

    R = 1000
    grid = N_NODES // R
    h_ext, aux_d = pl.pallas_call(
        _prep_body,
        grid=(grid,),
        in_specs=[
            pl.BlockSpec((R, IN_FEAT), lambda i: (i, 0)),
            pl.BlockSpec((IN_FEAT, HF), lambda i: (0, 0)),
            pl.BlockSpec((IN_FEAT, AUXW), lambda i: (0, 0)),
            pl.BlockSpec((IN_FEAT, AUXW), lambda i: (0, 0)),
            pl.BlockSpec((R, 1), lambda i: (i, 0)),
        ],
        out_specs=[
            pl.BlockSpec((R, ACCW), lambda i: (i, 0)),
            pl.BlockSpec((R, AUXW), lambda i: (i, 0)),
        ],
        out_shape=[
            jax.ShapeDtypeStruct((N_NODES, ACCW), f32),
            jax.ShapeDtypeStruct((N_NODES, AUXW), f32),
        ],
    )(x, wf, as_mat, ad_mat, tf)

    src = edge_index[0].reshape(NW, NCHUNK, CHUNK)
    dst = edge_index[1].reshape(NW, NCHUNK, CHUNK)
    zeros = jnp.zeros((ROWS_PER_TILE, ACCW), f32)
    ntw = jnp.full((16,), -temporal_weight, f32)

    mesh = plsc.VectorSubcoreMesh(core_axis_name="c", subcore_axis_name="s")
    partial = pl.kernel(
        _edge_body,
        out_type=jax.ShapeDtypeStruct((NC, N_NODES, ACCW), f32),
        mesh=mesh,
        compiler_params=pltpu.CompilerParams(use_tc_tiling_on_sc=False),
        scratch_types=[
            pltpu.VMEM_SHARED((N_NODES, ACCW), f32),
            pltpu.VMEM((4, CHUNK), jnp.int32),
            pltpu.VMEM((4, CHUNK), jnp.int32),
            pltpu.VMEM((2, CHUNK, ACCW), f32),
            pltpu.VMEM((2, CHUNK, AUXW), f32),
            pltpu.VMEM((2, CHUNK, ACCW), f32),
            pltpu.VMEM((16,), f32),
            pltpu.SemaphoreType.DMA((4,)),
            pltpu.SemaphoreType.DMA((2,)),
            pltpu.SemaphoreType.DMA((2,)),
        ],
    )(src, dst, h_ext, aux_d, zeros, ntw)

    sel = (eye[:, :, None] * jnp.ones((1, 1, OUT_FEAT), f32)).reshape(HEADS, HF)
    out = pl.pallas_call(
        _final_body,
        grid=(grid,),
        in_specs=[
            pl.BlockSpec((R, ACCW), lambda i: (i, 0)),
            pl.BlockSpec((R, ACCW), lambda i: (i, 0)),
            pl.BlockSpec((HEADS, HF), lambda i: (0, 0)),
        ],
        out_specs=pl.BlockSpec((R, HF), lambda i: (i, 0)),
        out_shape=jax.ShapeDtypeStruct((N_NODES, HF), f32),
    )(partial[0], partial[1], sel)
    return out
